# baseline (speedup 1.0000x reference)
import functools

import jax
import jax.numpy as jnp
from jax.experimental import pallas as pl
from jax.experimental.pallas import tpu as pltpu

_CHUNKS = [4000, 12000, 16000, 16000, 16000, 16000, 12000, 4000, 2000, 2000]
_IN_SLOTS = 3
_OUT_SLOTS = 2
_MAX_CHUNK = max(_CHUNKS)


def _outer(x_hbm, w_ref, b_ref, o_hbm, in_bufs, out_bufs, in_sems, out_sems):
    starts = []
    s = 0
    for c in _CHUNKS:
        starts.append(s)
        s += c
    n_chunks = len(_CHUNKS)

    def in_copy(idx):
        slot = idx % _IN_SLOTS
        return pltpu.make_async_copy(
            x_hbm.at[pl.ds(starts[idx], _CHUNKS[idx]), :],
            in_bufs.at[slot, pl.ds(0, _CHUNKS[idx]), :],
            in_sems.at[slot],
        )

    def out_copy(idx):
        slot = idx % _OUT_SLOTS
        return pltpu.make_async_copy(
            out_bufs.at[slot, pl.ds(0, _CHUNKS[idx]), :],
            o_hbm.at[pl.ds(starts[idx], _CHUNKS[idx]), :],
            out_sems.at[slot],
        )

    for idx in range(_IN_SLOTS):
        in_copy(idx).start()

    for idx in range(n_chunks):
        in_slot = idx % _IN_SLOTS
        out_slot = idx % _OUT_SLOTS
        in_copy(idx).wait()
        if idx >= _OUT_SLOTS:
            out_copy(idx - _OUT_SLOTS).wait()
        size = _CHUNKS[idx]
        h = jnp.dot(in_bufs[in_slot, :size, :], w_ref[...],
                    preferred_element_type=jnp.float32)
        out_bufs[out_slot, :size, :] = jnp.maximum(h + b_ref[...], 0.0)
        out_copy(idx).start()
        if idx + _IN_SLOTS < n_chunks:
            in_copy(idx + _IN_SLOTS).start()

    for idx in range(max(n_chunks - _OUT_SLOTS, 0), n_chunks):
        out_copy(idx).wait()


@functools.partial(jax.jit, static_argnames=())
def kernel(x, weights_encode, bias_encode):
    n, d_in = x.shape
    d_out = weights_encode.shape[1]
    bias2d = bias_encode.reshape(1, d_out)
    return pl.pallas_call(
        _outer,
        in_specs=[
            pl.BlockSpec(memory_space=pl.ANY),
            pl.BlockSpec(memory_space=pltpu.VMEM),
            pl.BlockSpec(memory_space=pltpu.VMEM),
        ],
        out_specs=pl.BlockSpec(memory_space=pl.ANY),
        out_shape=jax.ShapeDtypeStruct((n, d_out), jnp.float32),
        scratch_shapes=[
            pltpu.VMEM((_IN_SLOTS, _MAX_CHUNK, 128), jnp.float32),
            pltpu.VMEM((_OUT_SLOTS, _MAX_CHUNK, 128), jnp.float32),
            pltpu.SemaphoreType.DMA((_IN_SLOTS,)),
            pltpu.SemaphoreType.DMA((_OUT_SLOTS,)),
        ],
    )(x, weights_encode, bias2d)


# speedup vs baseline: 1.0438x; 1.0438x over previous
import functools

import jax
import jax.numpy as jnp
from jax.experimental import pallas as pl
from jax.experimental.pallas import tpu as pltpu

_CHUNKS = [8000, 16000, 20000, 20000, 20000, 12000, 4000]
_IN_SLOTS = 3
_OUT_SLOTS = 2
_MAX_CHUNK = max(_CHUNKS)


def _outer(x_hbm, w_ref, b_ref, o_hbm, in_bufs, out_bufs, in_sems, out_sems):
    starts = []
    s = 0
    for c in _CHUNKS:
        starts.append(s)
        s += c
    n_chunks = len(_CHUNKS)

    def in_copy(idx):
        slot = idx % _IN_SLOTS
        return pltpu.make_async_copy(
            x_hbm.at[pl.ds(starts[idx], _CHUNKS[idx]), :],
            in_bufs.at[slot, pl.ds(0, _CHUNKS[idx]), :],
            in_sems.at[slot],
        )

    def out_copy(idx):
        slot = idx % _OUT_SLOTS
        return pltpu.make_async_copy(
            out_bufs.at[slot, pl.ds(0, _CHUNKS[idx]), :],
            o_hbm.at[pl.ds(starts[idx], _CHUNKS[idx]), :],
            out_sems.at[slot],
        )

    for idx in range(_IN_SLOTS):
        in_copy(idx).start()

    for idx in range(n_chunks):
        in_slot = idx % _IN_SLOTS
        out_slot = idx % _OUT_SLOTS
        in_copy(idx).wait()
        if idx >= _OUT_SLOTS:
            out_copy(idx - _OUT_SLOTS).wait()
        size = _CHUNKS[idx]
        h = jnp.dot(in_bufs[in_slot, :size, :], w_ref[...],
                    preferred_element_type=jnp.float32)
        out_bufs[out_slot, :size, :] = jnp.maximum(h + b_ref[...], 0.0)
        out_copy(idx).start()
        if idx + _IN_SLOTS < n_chunks:
            in_copy(idx + _IN_SLOTS).start()

    for idx in range(max(n_chunks - _OUT_SLOTS, 0), n_chunks):
        out_copy(idx).wait()


@functools.partial(jax.jit, static_argnames=())
def kernel(x, weights_encode, bias_encode):
    n, d_in = x.shape
    d_out = weights_encode.shape[1]
    bias2d = bias_encode.reshape(1, d_out)
    return pl.pallas_call(
        _outer,
        in_specs=[
            pl.BlockSpec(memory_space=pl.ANY),
            pl.BlockSpec(memory_space=pltpu.VMEM),
            pl.BlockSpec(memory_space=pltpu.VMEM),
        ],
        out_specs=pl.BlockSpec(memory_space=pl.ANY),
        out_shape=jax.ShapeDtypeStruct((n, d_out), jnp.float32),
        scratch_shapes=[
            pltpu.VMEM((_IN_SLOTS, _MAX_CHUNK, 128), jnp.float32),
            pltpu.VMEM((_OUT_SLOTS, _MAX_CHUNK, 128), jnp.float32),
            pltpu.SemaphoreType.DMA((_IN_SLOTS,)),
            pltpu.SemaphoreType.DMA((_OUT_SLOTS,)),
        ],
    )(x, weights_encode, bias2d)
